# baseline (speedup 1.0000x reference)
import functools

import jax
import jax.numpy as jnp
from jax import lax
from jax.experimental import pallas as pl
from jax.experimental.pallas import tpu as pltpu
from jax.experimental.pallas import tpu_sc as plsc

N = 10000
E = 320000
D = 128

NC = 2
NS = 16
NW = NC * NS
CHUNK = 80
N_CH0 = 143
N_CH1 = 107
R_TILE = 632
R_LAST = N - (NS - 1) * R_TILE
N_ACC = NS * R_TILE


NBUF = 4
IBUF = 8
DI = 4
DG = 2


def _sc_scatter_body(t_hbm, edge_hbm, out_hbm,
                     sidx_v, didx_v, rows_v, acc_sh, gsem, isem, ssem):
    c = lax.axis_index("c")
    s = lax.axis_index("s")
    n_my = jnp.where(c == 0, N_CH0, N_CH1)
    base = jnp.where(c == 0, s * (N_CH0 * CHUNK),
                     NS * (N_CH0 * CHUNK) + s * (N_CH1 * CHUNK))

    def load_idx(j):
        slot = lax.rem(j, IBUF)
        off = base + j * CHUNK
        pltpu.async_copy(edge_hbm.at[pl.ds(off, CHUNK)], sidx_v.at[slot], isem)
        pltpu.async_copy(edge_hbm.at[pl.ds(E + off, CHUNK)], didx_v.at[slot], isem)

    def wait_idx(j):
        slot = lax.rem(j, IBUF)
        off = base + j * CHUNK
        pltpu.make_async_copy(edge_hbm.at[pl.ds(off, CHUNK)], sidx_v.at[slot], isem).wait()
        pltpu.make_async_copy(edge_hbm.at[pl.ds(E + off, CHUNK)], didx_v.at[slot], isem).wait()

    @pl.when(s < NS - 1)
    def _():
        pltpu.sync_copy(t_hbm.at[pl.ds(s * R_TILE, R_TILE)],
                        acc_sh.at[pl.ds(s * R_TILE, R_TILE)])

    @pl.when(s == NS - 1)
    def _():
        pltpu.sync_copy(t_hbm.at[pl.ds((NS - 1) * R_TILE, R_LAST)],
                        acc_sh.at[pl.ds((NS - 1) * R_TILE, R_LAST)])

    plsc.subcore_barrier()

    for j in range(DI):
        load_idx(j)
    for j in range(DG):
        wait_idx(j)
        pltpu.async_copy(t_hbm.at[sidx_v.at[j]], rows_v.at[j], gsem)

    def step(i, carry):
        b = lax.rem(i, NBUF)
        ib = lax.rem(i, IBUF)

        @pl.when(i + DI < n_my)
        def _():
            load_idx(i + DI)

        pltpu.make_async_copy(t_hbm.at[sidx_v.at[ib]], rows_v.at[b], gsem).wait()
        pltpu.async_copy(rows_v.at[b], acc_sh.at[didx_v.at[ib]], ssem, add=True)

        @pl.when(i >= NBUF - DG)
        def _():
            pltpu.make_async_copy(rows_v.at[b], acc_sh.at[didx_v.at[ib]],
                                  ssem).wait()

        @pl.when(i + DG < n_my)
        def _():
            wait_idx(i + DG)
            nib = lax.rem(i + DG, IBUF)
            nb = lax.rem(i + DG, NBUF)
            pltpu.async_copy(t_hbm.at[sidx_v.at[nib]], rows_v.at[nb], gsem)

        return carry

    lax.fori_loop(0, n_my, step, 0)
    for j in range(NBUF - DG):
        pltpu.make_async_copy(rows_v.at[j], acc_sh.at[didx_v.at[0]], ssem).wait()
    plsc.subcore_barrier()

    @pl.when(s < NS - 1)
    def _():
        pltpu.sync_copy(acc_sh.at[pl.ds(s * R_TILE, R_TILE)],
                        out_hbm.at[c, pl.ds(s * R_TILE, R_TILE)])

    @pl.when(s == NS - 1)
    def _():
        pltpu.sync_copy(acc_sh.at[pl.ds((NS - 1) * R_TILE, R_LAST)],
                        out_hbm.at[c, pl.ds((NS - 1) * R_TILE, R_LAST)])


_sc_scatter = functools.partial(
    pl.kernel,
    out_type=jax.ShapeDtypeStruct((NC, N, D), jnp.float32),
    mesh=plsc.VectorSubcoreMesh(core_axis_name="c", subcore_axis_name="s"),
    scratch_types=[
        pltpu.VMEM((IBUF, CHUNK), jnp.int32),
        pltpu.VMEM((IBUF, CHUNK), jnp.int32),
        pltpu.VMEM((NBUF, CHUNK, D), jnp.float32),
        pltpu.VMEM_SHARED((N_ACC, D), jnp.float32),
        pltpu.SemaphoreType.DMA,
        pltpu.SemaphoreType.DMA,
        pltpu.SemaphoreType.DMA,
    ],
)(_sc_scatter_body)


def _weights_body(w1_ref, b1_ref, w2_ref, b2_ref, w_ref, c_ref):
    w_ref[...] = lax.dot_general(w1_ref[...], w2_ref[...],
                                 (((0,), (1,)), ((), ())),
                                 preferred_element_type=jnp.float32)
    c_ref[...] = lax.dot_general(b1_ref[...], w2_ref[...],
                                 (((1,), (1,)), ((), ())),
                                 preferred_element_type=jnp.float32) + b2_ref[...]


def _combine_weights(W1, b1, W2, b2):
    return pl.pallas_call(
        _weights_body,
        out_shape=(jax.ShapeDtypeStruct((D, D), jnp.float32),
                   jax.ShapeDtypeStruct((1, D), jnp.float32)),
    )(W1, b1[None, :], W2, b2[None, :])


_BLK = 2000
_GRID = N // _BLK


def _mm1_body(x_ref, w_ref, c_ref, o_ref):
    o_ref[...] = jnp.dot(x_ref[...], w_ref[...],
                         preferred_element_type=jnp.float32) + c_ref[...]


def _mm1(x, W, c):
    return pl.pallas_call(
        _mm1_body,
        grid=(_GRID,),
        in_specs=[pl.BlockSpec((_BLK, D), lambda i: (i, 0)),
                  pl.BlockSpec((D, D), lambda i: (0, 0)),
                  pl.BlockSpec((1, D), lambda i: (0, 0))],
        out_specs=pl.BlockSpec((_BLK, D), lambda i: (i, 0)),
        out_shape=jax.ShapeDtypeStruct((N, D), jnp.float32),
    )(x, W, c)


def _mm2_body(p_ref, t_ref, w_ref, c_ref, o_ref):
    h = p_ref[0] + p_ref[1] - t_ref[...]
    o_ref[...] = jnp.dot(h, w_ref[...],
                         preferred_element_type=jnp.float32) + c_ref[...]


def _mm2(parts, t, W, c):
    return pl.pallas_call(
        _mm2_body,
        grid=(_GRID,),
        in_specs=[pl.BlockSpec((NC, _BLK, D), lambda i: (0, i, 0)),
                  pl.BlockSpec((_BLK, D), lambda i: (i, 0)),
                  pl.BlockSpec((D, D), lambda i: (0, 0)),
                  pl.BlockSpec((1, D), lambda i: (0, 0))],
        out_specs=pl.BlockSpec((_BLK, D), lambda i: (i, 0)),
        out_shape=jax.ShapeDtypeStruct((N, D), jnp.float32),
    )(parts, t, W, c)


def _final_body(p_ref, t_ref, o_ref):
    z = p_ref[0] + p_ref[1] - t_ref[...]
    m = jnp.max(z, axis=1, keepdims=True)
    e = jnp.exp(z - m)
    o_ref[...] = (z - m) - jnp.log(jnp.sum(e, axis=1, keepdims=True))


def _final(parts, t):
    return pl.pallas_call(
        _final_body,
        grid=(_GRID,),
        in_specs=[pl.BlockSpec((NC, _BLK, D), lambda i: (0, i, 0)),
                  pl.BlockSpec((_BLK, D), lambda i: (i, 0))],
        out_specs=pl.BlockSpec((_BLK, D), lambda i: (i, 0)),
        out_shape=jax.ShapeDtypeStruct((N, D), jnp.float32),
    )(parts, t)


def kernel(x, edge_index, W1, b1, W2, b2):
    W, c = _combine_weights(W1, b1, W2, b2)
    t1 = _mm1(x, W, c)
    edge_flat = edge_index.reshape(2 * E)
    parts1 = _sc_scatter(t1, edge_flat)
    t2 = _mm2(parts1, t1, W, c)
    parts2 = _sc_scatter(t2, edge_flat)
    return _final(parts2, t2)


# speedup vs baseline: 1.0951x; 1.0951x over previous
import functools

import jax
import jax.numpy as jnp
from jax import lax
from jax.experimental import pallas as pl
from jax.experimental.pallas import tpu as pltpu
from jax.experimental.pallas import tpu_sc as plsc

N = 10000
E = 320000
D = 128

NC = 2
NS = 16
NW = NC * NS
CHUNK = 80
E_TILE = E // NW
N_CH = E_TILE // CHUNK
R_TILE = 632
R_LAST = N - (NS - 1) * R_TILE
N_ACC = NS * R_TILE


NBUF = 4
IBUF = 8
DI = 4
DG = 2


def _sc_scatter_body(t_hbm, edge_hbm, out_hbm,
                     sidx_v, didx_v, rows_v, acc_sh, gsem, isem, ssem):
    c = lax.axis_index("c")
    s = lax.axis_index("s")
    wid = s * NC + c
    base = wid * E_TILE

    def load_idx(j):
        slot = lax.rem(j, IBUF)
        off = base + j * CHUNK
        pltpu.async_copy(edge_hbm.at[pl.ds(off, CHUNK)], sidx_v.at[slot], isem)
        pltpu.async_copy(edge_hbm.at[pl.ds(E + off, CHUNK)], didx_v.at[slot], isem)

    def wait_idx(j):
        slot = lax.rem(j, IBUF)
        off = base + j * CHUNK
        pltpu.make_async_copy(edge_hbm.at[pl.ds(off, CHUNK)], sidx_v.at[slot], isem).wait()
        pltpu.make_async_copy(edge_hbm.at[pl.ds(E + off, CHUNK)], didx_v.at[slot], isem).wait()

    @pl.when(s < NS - 1)
    def _():
        pltpu.sync_copy(t_hbm.at[pl.ds(s * R_TILE, R_TILE)],
                        acc_sh.at[pl.ds(s * R_TILE, R_TILE)])

    @pl.when(s == NS - 1)
    def _():
        pltpu.sync_copy(t_hbm.at[pl.ds((NS - 1) * R_TILE, R_LAST)],
                        acc_sh.at[pl.ds((NS - 1) * R_TILE, R_LAST)])

    plsc.subcore_barrier()

    for j in range(DI):
        load_idx(j)
    for j in range(DG):
        wait_idx(j)
        pltpu.async_copy(t_hbm.at[sidx_v.at[j]], rows_v.at[j], gsem)

    def step(i, carry):
        b = lax.rem(i, NBUF)
        ib = lax.rem(i, IBUF)

        @pl.when(i + DI < N_CH)
        def _():
            load_idx(i + DI)

        pltpu.make_async_copy(t_hbm.at[sidx_v.at[ib]], rows_v.at[b], gsem).wait()
        pltpu.async_copy(rows_v.at[b], acc_sh.at[didx_v.at[ib]], ssem, add=True)

        @pl.when(i >= NBUF - DG)
        def _():
            pltpu.make_async_copy(rows_v.at[b], acc_sh.at[didx_v.at[ib]],
                                  ssem).wait()

        @pl.when(i + DG < N_CH)
        def _():
            wait_idx(i + DG)
            nib = lax.rem(i + DG, IBUF)
            nb = lax.rem(i + DG, NBUF)
            pltpu.async_copy(t_hbm.at[sidx_v.at[nib]], rows_v.at[nb], gsem)

        return carry

    lax.fori_loop(0, N_CH, step, 0)
    for j in range(NBUF - DG):
        pltpu.make_async_copy(rows_v.at[j], acc_sh.at[didx_v.at[0]], ssem).wait()
    plsc.subcore_barrier()

    @pl.when(s < NS - 1)
    def _():
        pltpu.sync_copy(acc_sh.at[pl.ds(s * R_TILE, R_TILE)],
                        out_hbm.at[c, pl.ds(s * R_TILE, R_TILE)])

    @pl.when(s == NS - 1)
    def _():
        pltpu.sync_copy(acc_sh.at[pl.ds((NS - 1) * R_TILE, R_LAST)],
                        out_hbm.at[c, pl.ds((NS - 1) * R_TILE, R_LAST)])


_sc_scatter = functools.partial(
    pl.kernel,
    out_type=jax.ShapeDtypeStruct((NC, N, D), jnp.float32),
    mesh=plsc.VectorSubcoreMesh(core_axis_name="c", subcore_axis_name="s"),
    scratch_types=[
        pltpu.VMEM((IBUF, CHUNK), jnp.int32),
        pltpu.VMEM((IBUF, CHUNK), jnp.int32),
        pltpu.VMEM((NBUF, CHUNK, D), jnp.float32),
        pltpu.VMEM_SHARED((N_ACC, D), jnp.float32),
        pltpu.SemaphoreType.DMA,
        pltpu.SemaphoreType.DMA,
        pltpu.SemaphoreType.DMA,
    ],
)(_sc_scatter_body)


def _weights_body(w1_ref, b1_ref, w2_ref, b2_ref, w_ref, c_ref):
    w_ref[...] = lax.dot_general(w1_ref[...], w2_ref[...],
                                 (((0,), (1,)), ((), ())),
                                 preferred_element_type=jnp.float32)
    c_ref[...] = lax.dot_general(b1_ref[...], w2_ref[...],
                                 (((1,), (1,)), ((), ())),
                                 preferred_element_type=jnp.float32) + b2_ref[...]


def _combine_weights(W1, b1, W2, b2):
    return pl.pallas_call(
        _weights_body,
        out_shape=(jax.ShapeDtypeStruct((D, D), jnp.float32),
                   jax.ShapeDtypeStruct((1, D), jnp.float32)),
    )(W1, b1[None, :], W2, b2[None, :])


_BLK = 2000
_GRID = N // _BLK


def _mm1_body(x_ref, w_ref, c_ref, o_ref):
    o_ref[...] = jnp.dot(x_ref[...], w_ref[...],
                         preferred_element_type=jnp.float32) + c_ref[...]


def _mm1(x, W, c):
    return pl.pallas_call(
        _mm1_body,
        grid=(_GRID,),
        in_specs=[pl.BlockSpec((_BLK, D), lambda i: (i, 0)),
                  pl.BlockSpec((D, D), lambda i: (0, 0)),
                  pl.BlockSpec((1, D), lambda i: (0, 0))],
        out_specs=pl.BlockSpec((_BLK, D), lambda i: (i, 0)),
        out_shape=jax.ShapeDtypeStruct((N, D), jnp.float32),
    )(x, W, c)


def _mm2_body(p_ref, t_ref, w_ref, c_ref, o_ref):
    h = p_ref[0] + p_ref[1] - t_ref[...]
    o_ref[...] = jnp.dot(h, w_ref[...],
                         preferred_element_type=jnp.float32) + c_ref[...]


def _mm2(parts, t, W, c):
    return pl.pallas_call(
        _mm2_body,
        grid=(_GRID,),
        in_specs=[pl.BlockSpec((NC, _BLK, D), lambda i: (0, i, 0)),
                  pl.BlockSpec((_BLK, D), lambda i: (i, 0)),
                  pl.BlockSpec((D, D), lambda i: (0, 0)),
                  pl.BlockSpec((1, D), lambda i: (0, 0))],
        out_specs=pl.BlockSpec((_BLK, D), lambda i: (i, 0)),
        out_shape=jax.ShapeDtypeStruct((N, D), jnp.float32),
    )(parts, t, W, c)


def _final_body(p_ref, t_ref, o_ref):
    z = p_ref[0] + p_ref[1] - t_ref[...]
    m = jnp.max(z, axis=1, keepdims=True)
    e = jnp.exp(z - m)
    o_ref[...] = (z - m) - jnp.log(jnp.sum(e, axis=1, keepdims=True))


def _final(parts, t):
    return pl.pallas_call(
        _final_body,
        grid=(_GRID,),
        in_specs=[pl.BlockSpec((NC, _BLK, D), lambda i: (0, i, 0)),
                  pl.BlockSpec((_BLK, D), lambda i: (i, 0))],
        out_specs=pl.BlockSpec((_BLK, D), lambda i: (i, 0)),
        out_shape=jax.ShapeDtypeStruct((N, D), jnp.float32),
    )(parts, t)


def kernel(x, edge_index, W1, b1, W2, b2):
    W, c = _combine_weights(W1, b1, W2, b2)
    t1 = _mm1(x, W, c)
    edge_flat = edge_index.reshape(2 * E)
    parts1 = _sc_scatter(t1, edge_flat)
    t2 = _mm2(parts1, t1, W, c)
    parts2 = _sc_scatter(t2, edge_flat)
    return _final(parts2, t2)


# speedup vs baseline: 1.1306x; 1.0324x over previous
import functools

import jax
import jax.numpy as jnp
from jax import lax
from jax.experimental import pallas as pl
from jax.experimental.pallas import tpu as pltpu
from jax.experimental.pallas import tpu_sc as plsc

N = 10000
E = 320000
D = 128

NC = 2
NS = 16
NW = NC * NS
CHUNK = 40
E_TILE = E // NW
N_CH = E_TILE // CHUNK
R_TILE = 632
R_LAST = N - (NS - 1) * R_TILE
N_ACC = NS * R_TILE


NBUF = 8
IBUF = 8
DI = 6
DG = 5


def _sc_scatter_body(t_hbm, edge_hbm, out_hbm,
                     sidx_v, didx_v, rows_v, acc_sh, gsem, isem, ssem):
    c = lax.axis_index("c")
    s = lax.axis_index("s")
    wid = s * NC + c
    base = wid * E_TILE

    def load_idx(j):
        slot = lax.rem(j, IBUF)
        off = base + j * CHUNK
        pltpu.async_copy(edge_hbm.at[pl.ds(off, CHUNK)], sidx_v.at[slot], isem)
        pltpu.async_copy(edge_hbm.at[pl.ds(E + off, CHUNK)], didx_v.at[slot], isem)

    def wait_idx(j):
        slot = lax.rem(j, IBUF)
        off = base + j * CHUNK
        pltpu.make_async_copy(edge_hbm.at[pl.ds(off, CHUNK)], sidx_v.at[slot], isem).wait()
        pltpu.make_async_copy(edge_hbm.at[pl.ds(E + off, CHUNK)], didx_v.at[slot], isem).wait()

    @pl.when(s < NS - 1)
    def _():
        pltpu.sync_copy(t_hbm.at[pl.ds(s * R_TILE, R_TILE)],
                        acc_sh.at[pl.ds(s * R_TILE, R_TILE)])

    @pl.when(s == NS - 1)
    def _():
        pltpu.sync_copy(t_hbm.at[pl.ds((NS - 1) * R_TILE, R_LAST)],
                        acc_sh.at[pl.ds((NS - 1) * R_TILE, R_LAST)])

    plsc.subcore_barrier()

    for j in range(DI):
        load_idx(j)
    for j in range(DG):
        wait_idx(j)
        pltpu.async_copy(t_hbm.at[sidx_v.at[j]], rows_v.at[j], gsem)

    def step(i, carry):
        b = lax.rem(i, NBUF)
        ib = lax.rem(i, IBUF)

        @pl.when(i + DI < N_CH)
        def _():
            load_idx(i + DI)

        pltpu.make_async_copy(t_hbm.at[sidx_v.at[ib]], rows_v.at[b], gsem).wait()
        pltpu.async_copy(rows_v.at[b], acc_sh.at[didx_v.at[ib]], ssem, add=True)

        @pl.when(i >= NBUF - DG)
        def _():
            pltpu.make_async_copy(rows_v.at[b], acc_sh.at[didx_v.at[ib]],
                                  ssem).wait()

        @pl.when(i + DG < N_CH)
        def _():
            wait_idx(i + DG)
            nib = lax.rem(i + DG, IBUF)
            nb = lax.rem(i + DG, NBUF)
            pltpu.async_copy(t_hbm.at[sidx_v.at[nib]], rows_v.at[nb], gsem)

        return carry

    lax.fori_loop(0, N_CH, step, 0)
    for j in range(NBUF - DG):
        pltpu.make_async_copy(rows_v.at[j], acc_sh.at[didx_v.at[0]], ssem).wait()
    plsc.subcore_barrier()

    @pl.when(s < NS - 1)
    def _():
        pltpu.sync_copy(acc_sh.at[pl.ds(s * R_TILE, R_TILE)],
                        out_hbm.at[c, pl.ds(s * R_TILE, R_TILE)])

    @pl.when(s == NS - 1)
    def _():
        pltpu.sync_copy(acc_sh.at[pl.ds((NS - 1) * R_TILE, R_LAST)],
                        out_hbm.at[c, pl.ds((NS - 1) * R_TILE, R_LAST)])


_sc_scatter = functools.partial(
    pl.kernel,
    out_type=jax.ShapeDtypeStruct((NC, N, D), jnp.float32),
    mesh=plsc.VectorSubcoreMesh(core_axis_name="c", subcore_axis_name="s"),
    scratch_types=[
        pltpu.VMEM((IBUF, CHUNK), jnp.int32),
        pltpu.VMEM((IBUF, CHUNK), jnp.int32),
        pltpu.VMEM((NBUF, CHUNK, D), jnp.float32),
        pltpu.VMEM_SHARED((N_ACC, D), jnp.float32),
        pltpu.SemaphoreType.DMA,
        pltpu.SemaphoreType.DMA,
        pltpu.SemaphoreType.DMA,
    ],
)(_sc_scatter_body)


def _weights_body(w1_ref, b1_ref, w2_ref, b2_ref, w_ref, c_ref):
    w_ref[...] = lax.dot_general(w1_ref[...], w2_ref[...],
                                 (((0,), (1,)), ((), ())),
                                 preferred_element_type=jnp.float32)
    c_ref[...] = lax.dot_general(b1_ref[...], w2_ref[...],
                                 (((1,), (1,)), ((), ())),
                                 preferred_element_type=jnp.float32) + b2_ref[...]


def _combine_weights(W1, b1, W2, b2):
    return pl.pallas_call(
        _weights_body,
        out_shape=(jax.ShapeDtypeStruct((D, D), jnp.float32),
                   jax.ShapeDtypeStruct((1, D), jnp.float32)),
    )(W1, b1[None, :], W2, b2[None, :])


_BLK = 2000
_GRID = N // _BLK


def _mm1_body(x_ref, w_ref, c_ref, o_ref):
    o_ref[...] = jnp.dot(x_ref[...], w_ref[...],
                         preferred_element_type=jnp.float32) + c_ref[...]


def _mm1(x, W, c):
    return pl.pallas_call(
        _mm1_body,
        grid=(_GRID,),
        in_specs=[pl.BlockSpec((_BLK, D), lambda i: (i, 0)),
                  pl.BlockSpec((D, D), lambda i: (0, 0)),
                  pl.BlockSpec((1, D), lambda i: (0, 0))],
        out_specs=pl.BlockSpec((_BLK, D), lambda i: (i, 0)),
        out_shape=jax.ShapeDtypeStruct((N, D), jnp.float32),
    )(x, W, c)


def _mm2_body(p_ref, t_ref, w_ref, c_ref, o_ref):
    h = p_ref[0] + p_ref[1] - t_ref[...]
    o_ref[...] = jnp.dot(h, w_ref[...],
                         preferred_element_type=jnp.float32) + c_ref[...]


def _mm2(parts, t, W, c):
    return pl.pallas_call(
        _mm2_body,
        grid=(_GRID,),
        in_specs=[pl.BlockSpec((NC, _BLK, D), lambda i: (0, i, 0)),
                  pl.BlockSpec((_BLK, D), lambda i: (i, 0)),
                  pl.BlockSpec((D, D), lambda i: (0, 0)),
                  pl.BlockSpec((1, D), lambda i: (0, 0))],
        out_specs=pl.BlockSpec((_BLK, D), lambda i: (i, 0)),
        out_shape=jax.ShapeDtypeStruct((N, D), jnp.float32),
    )(parts, t, W, c)


def _final_body(p_ref, t_ref, o_ref):
    z = p_ref[0] + p_ref[1] - t_ref[...]
    m = jnp.max(z, axis=1, keepdims=True)
    e = jnp.exp(z - m)
    o_ref[...] = (z - m) - jnp.log(jnp.sum(e, axis=1, keepdims=True))


def _final(parts, t):
    return pl.pallas_call(
        _final_body,
        grid=(_GRID,),
        in_specs=[pl.BlockSpec((NC, _BLK, D), lambda i: (0, i, 0)),
                  pl.BlockSpec((_BLK, D), lambda i: (i, 0))],
        out_specs=pl.BlockSpec((_BLK, D), lambda i: (i, 0)),
        out_shape=jax.ShapeDtypeStruct((N, D), jnp.float32),
    )(parts, t)


def kernel(x, edge_index, W1, b1, W2, b2):
    W, c = _combine_weights(W1, b1, W2, b2)
    t1 = _mm1(x, W, c)
    edge_flat = edge_index.reshape(2 * E)
    parts1 = _sc_scatter(t1, edge_flat)
    t2 = _mm2(parts1, t1, W, c)
    parts2 = _sc_scatter(t2, edge_flat)
    return _final(parts2, t2)


# speedup vs baseline: 1.1554x; 1.0219x over previous
import functools

import jax
import jax.numpy as jnp
from jax import lax
from jax.experimental import pallas as pl
from jax.experimental.pallas import tpu as pltpu
from jax.experimental.pallas import tpu_sc as plsc

N = 10000
E = 320000
D = 128

NC = 2
NS = 16
NW = NC * NS
CHUNK = 40
E_TILE = E // NW
N_CH = E_TILE // CHUNK
R_TILE = 632
R_LAST = N - (NS - 1) * R_TILE
N_ACC = NS * R_TILE


NBUF = 8
IBUF = 8
DI = 6
DG = 5


def _sc_scatter_body(t_hbm, edge_hbm, out_hbm,
                     sidx_v, didx_v, rows_v, acc_sh, gsem, isem, ssem, nsem):
    c = lax.axis_index("c")
    s = lax.axis_index("s")
    wid = s * NC + c
    base = wid * E_TILE

    def load_idx(j):
        slot = lax.rem(j, IBUF)
        off = base + j * CHUNK
        pltpu.async_copy(edge_hbm.at[pl.ds(off, CHUNK)], sidx_v.at[slot], isem)
        pltpu.async_copy(edge_hbm.at[pl.ds(E + off, CHUNK)], didx_v.at[slot], isem)

    def wait_idx(j):
        slot = lax.rem(j, IBUF)
        off = base + j * CHUNK
        pltpu.make_async_copy(edge_hbm.at[pl.ds(off, CHUNK)], sidx_v.at[slot], isem).wait()
        pltpu.make_async_copy(edge_hbm.at[pl.ds(E + off, CHUNK)], didx_v.at[slot], isem).wait()

    @pl.when(s < NS - 1)
    def _():
        pltpu.async_copy(t_hbm.at[pl.ds(s * R_TILE, R_TILE)],
                         acc_sh.at[pl.ds(s * R_TILE, R_TILE)], nsem)

    @pl.when(s == NS - 1)
    def _():
        pltpu.async_copy(t_hbm.at[pl.ds((NS - 1) * R_TILE, R_LAST)],
                         acc_sh.at[pl.ds((NS - 1) * R_TILE, R_LAST)], nsem)

    for j in range(DI):
        load_idx(j)
    for j in range(DG):
        wait_idx(j)
        pltpu.async_copy(t_hbm.at[sidx_v.at[j]], rows_v.at[j], gsem)

    @pl.when(s < NS - 1)
    def _():
        pltpu.make_async_copy(t_hbm.at[pl.ds(s * R_TILE, R_TILE)],
                              acc_sh.at[pl.ds(s * R_TILE, R_TILE)], nsem).wait()

    @pl.when(s == NS - 1)
    def _():
        pltpu.make_async_copy(t_hbm.at[pl.ds((NS - 1) * R_TILE, R_LAST)],
                              acc_sh.at[pl.ds((NS - 1) * R_TILE, R_LAST)],
                              nsem).wait()

    plsc.subcore_barrier()

    def step(i, carry):
        b = lax.rem(i, NBUF)
        ib = lax.rem(i, IBUF)

        @pl.when(i + DI < N_CH)
        def _():
            load_idx(i + DI)

        pltpu.make_async_copy(t_hbm.at[sidx_v.at[ib]], rows_v.at[b], gsem).wait()
        pltpu.async_copy(rows_v.at[b], acc_sh.at[didx_v.at[ib]], ssem, add=True)

        @pl.when(i >= NBUF - DG)
        def _():
            pltpu.make_async_copy(rows_v.at[b], acc_sh.at[didx_v.at[ib]],
                                  ssem).wait()

        @pl.when(i + DG < N_CH)
        def _():
            wait_idx(i + DG)
            nib = lax.rem(i + DG, IBUF)
            nb = lax.rem(i + DG, NBUF)
            pltpu.async_copy(t_hbm.at[sidx_v.at[nib]], rows_v.at[nb], gsem)

        return carry

    lax.fori_loop(0, N_CH, step, 0)
    for j in range(NBUF - DG):
        pltpu.make_async_copy(rows_v.at[j], acc_sh.at[didx_v.at[0]], ssem).wait()
    plsc.subcore_barrier()

    @pl.when(s < NS - 1)
    def _():
        pltpu.sync_copy(acc_sh.at[pl.ds(s * R_TILE, R_TILE)],
                        out_hbm.at[c, pl.ds(s * R_TILE, R_TILE)])

    @pl.when(s == NS - 1)
    def _():
        pltpu.sync_copy(acc_sh.at[pl.ds((NS - 1) * R_TILE, R_LAST)],
                        out_hbm.at[c, pl.ds((NS - 1) * R_TILE, R_LAST)])


_sc_scatter = functools.partial(
    pl.kernel,
    out_type=jax.ShapeDtypeStruct((NC, N, D), jnp.float32),
    mesh=plsc.VectorSubcoreMesh(core_axis_name="c", subcore_axis_name="s"),
    scratch_types=[
        pltpu.VMEM((IBUF, CHUNK), jnp.int32),
        pltpu.VMEM((IBUF, CHUNK), jnp.int32),
        pltpu.VMEM((NBUF, CHUNK, D), jnp.float32),
        pltpu.VMEM_SHARED((N_ACC, D), jnp.float32),
        pltpu.SemaphoreType.DMA,
        pltpu.SemaphoreType.DMA,
        pltpu.SemaphoreType.DMA,
        pltpu.SemaphoreType.DMA,
    ],
)(_sc_scatter_body)


def _wc(w1_ref, b1_ref, w2_ref, b2_ref):
    W = lax.dot_general(w1_ref[...], w2_ref[...], (((0,), (1,)), ((), ())),
                        preferred_element_type=jnp.float32)
    cv = lax.dot_general(b1_ref[...], w2_ref[...], (((1,), (1,)), ((), ())),
                         preferred_element_type=jnp.float32) + b2_ref[...]
    return W, cv


_W_SPECS = [pl.BlockSpec((D, D), lambda i: (0, 0)),
            pl.BlockSpec((1, D), lambda i: (0, 0)),
            pl.BlockSpec((D, D), lambda i: (0, 0)),
            pl.BlockSpec((1, D), lambda i: (0, 0))]


_BLK = 2000
_GRID = N // _BLK


def _mm1_body(x_ref, w1_ref, b1_ref, w2_ref, b2_ref, o_ref):
    W, cv = _wc(w1_ref, b1_ref, w2_ref, b2_ref)
    o_ref[...] = jnp.dot(x_ref[...], W,
                         preferred_element_type=jnp.float32) + cv


def _mm1(x, W1, b1, W2, b2):
    return pl.pallas_call(
        _mm1_body,
        grid=(_GRID,),
        in_specs=[pl.BlockSpec((_BLK, D), lambda i: (i, 0))] + _W_SPECS,
        out_specs=pl.BlockSpec((_BLK, D), lambda i: (i, 0)),
        out_shape=jax.ShapeDtypeStruct((N, D), jnp.float32),
    )(x, W1, b1[None, :], W2, b2[None, :])


def _mm2_body(p_ref, t_ref, w1_ref, b1_ref, w2_ref, b2_ref, o_ref):
    W, cv = _wc(w1_ref, b1_ref, w2_ref, b2_ref)
    h = p_ref[0] + p_ref[1] - t_ref[...]
    o_ref[...] = jnp.dot(h, W, preferred_element_type=jnp.float32) + cv


def _mm2(parts, t, W1, b1, W2, b2):
    return pl.pallas_call(
        _mm2_body,
        grid=(_GRID,),
        in_specs=[pl.BlockSpec((NC, _BLK, D), lambda i: (0, i, 0)),
                  pl.BlockSpec((_BLK, D), lambda i: (i, 0))] + _W_SPECS,
        out_specs=pl.BlockSpec((_BLK, D), lambda i: (i, 0)),
        out_shape=jax.ShapeDtypeStruct((N, D), jnp.float32),
    )(parts, t, W1, b1[None, :], W2, b2[None, :])


def _final_body(p_ref, t_ref, o_ref):
    z = p_ref[0] + p_ref[1] - t_ref[...]
    m = jnp.max(z, axis=1, keepdims=True)
    e = jnp.exp(z - m)
    o_ref[...] = (z - m) - jnp.log(jnp.sum(e, axis=1, keepdims=True))


def _final(parts, t):
    return pl.pallas_call(
        _final_body,
        grid=(_GRID,),
        in_specs=[pl.BlockSpec((NC, _BLK, D), lambda i: (0, i, 0)),
                  pl.BlockSpec((_BLK, D), lambda i: (i, 0))],
        out_specs=pl.BlockSpec((_BLK, D), lambda i: (i, 0)),
        out_shape=jax.ShapeDtypeStruct((N, D), jnp.float32),
    )(parts, t)


def kernel(x, edge_index, W1, b1, W2, b2):
    t1 = _mm1(x, W1, b1, W2, b2)
    edge_flat = edge_index.reshape(2 * E)
    parts1 = _sc_scatter(t1, edge_flat)
    t2 = _mm2(parts1, t1, W1, b1, W2, b2)
    parts2 = _sc_scatter(t2, edge_flat)
    return _final(parts2, t2)


# speedup vs baseline: 1.1585x; 1.0027x over previous
import functools

import jax
import jax.numpy as jnp
from jax import lax
from jax.experimental import pallas as pl
from jax.experimental.pallas import tpu as pltpu
from jax.experimental.pallas import tpu_sc as plsc

N = 10000
E = 320000
D = 128

NC = 2
NS = 16
NW = NC * NS
CHUNK = 40
E_TILE = E // NW
N_CH = E_TILE // CHUNK
R_TILE = 632
R_LAST = N - (NS - 1) * R_TILE
N_ACC = NS * R_TILE


NBUF = 8
IBUF = 8
DI = 7
DG = 6


def _sc_scatter_body(t_hbm, edge_hbm, out_hbm,
                     sidx_v, didx_v, rows_v, acc_sh, gsem, isem, ssem, nsem):
    c = lax.axis_index("c")
    s = lax.axis_index("s")
    wid = s * NC + c
    base = wid * E_TILE

    def load_idx(j):
        slot = lax.rem(j, IBUF)
        off = base + j * CHUNK
        pltpu.async_copy(edge_hbm.at[pl.ds(off, CHUNK)], sidx_v.at[slot], isem)
        pltpu.async_copy(edge_hbm.at[pl.ds(E + off, CHUNK)], didx_v.at[slot], isem)

    def wait_idx(j):
        slot = lax.rem(j, IBUF)
        off = base + j * CHUNK
        pltpu.make_async_copy(edge_hbm.at[pl.ds(off, CHUNK)], sidx_v.at[slot], isem).wait()
        pltpu.make_async_copy(edge_hbm.at[pl.ds(E + off, CHUNK)], didx_v.at[slot], isem).wait()

    @pl.when(s < NS - 1)
    def _():
        pltpu.async_copy(t_hbm.at[pl.ds(s * R_TILE, R_TILE)],
                         acc_sh.at[pl.ds(s * R_TILE, R_TILE)], nsem)

    @pl.when(s == NS - 1)
    def _():
        pltpu.async_copy(t_hbm.at[pl.ds((NS - 1) * R_TILE, R_LAST)],
                         acc_sh.at[pl.ds((NS - 1) * R_TILE, R_LAST)], nsem)

    for j in range(DI):
        load_idx(j)
    for j in range(DG):
        wait_idx(j)
        pltpu.async_copy(t_hbm.at[sidx_v.at[j]], rows_v.at[j], gsem)

    @pl.when(s < NS - 1)
    def _():
        pltpu.make_async_copy(t_hbm.at[pl.ds(s * R_TILE, R_TILE)],
                              acc_sh.at[pl.ds(s * R_TILE, R_TILE)], nsem).wait()

    @pl.when(s == NS - 1)
    def _():
        pltpu.make_async_copy(t_hbm.at[pl.ds((NS - 1) * R_TILE, R_LAST)],
                              acc_sh.at[pl.ds((NS - 1) * R_TILE, R_LAST)],
                              nsem).wait()

    plsc.subcore_barrier()

    def step(i, carry):
        b = lax.rem(i, NBUF)
        ib = lax.rem(i, IBUF)

        @pl.when(i + DI < N_CH)
        def _():
            load_idx(i + DI)

        pltpu.make_async_copy(t_hbm.at[sidx_v.at[ib]], rows_v.at[b], gsem).wait()
        pltpu.async_copy(rows_v.at[b], acc_sh.at[didx_v.at[ib]], ssem, add=True)

        @pl.when(i >= NBUF - DG)
        def _():
            pltpu.make_async_copy(rows_v.at[b], acc_sh.at[didx_v.at[ib]],
                                  ssem).wait()

        @pl.when(i + DG < N_CH)
        def _():
            wait_idx(i + DG)
            nib = lax.rem(i + DG, IBUF)
            nb = lax.rem(i + DG, NBUF)
            pltpu.async_copy(t_hbm.at[sidx_v.at[nib]], rows_v.at[nb], gsem)

        return carry

    lax.fori_loop(0, N_CH, step, 0)
    for j in range(NBUF - DG):
        pltpu.make_async_copy(rows_v.at[j], acc_sh.at[didx_v.at[0]], ssem).wait()
    plsc.subcore_barrier()

    @pl.when(s < NS - 1)
    def _():
        pltpu.sync_copy(acc_sh.at[pl.ds(s * R_TILE, R_TILE)],
                        out_hbm.at[c, pl.ds(s * R_TILE, R_TILE)])

    @pl.when(s == NS - 1)
    def _():
        pltpu.sync_copy(acc_sh.at[pl.ds((NS - 1) * R_TILE, R_LAST)],
                        out_hbm.at[c, pl.ds((NS - 1) * R_TILE, R_LAST)])


_sc_scatter = functools.partial(
    pl.kernel,
    out_type=jax.ShapeDtypeStruct((NC, N, D), jnp.float32),
    mesh=plsc.VectorSubcoreMesh(core_axis_name="c", subcore_axis_name="s"),
    scratch_types=[
        pltpu.VMEM((IBUF, CHUNK), jnp.int32),
        pltpu.VMEM((IBUF, CHUNK), jnp.int32),
        pltpu.VMEM((NBUF, CHUNK, D), jnp.float32),
        pltpu.VMEM_SHARED((N_ACC, D), jnp.float32),
        pltpu.SemaphoreType.DMA,
        pltpu.SemaphoreType.DMA,
        pltpu.SemaphoreType.DMA,
        pltpu.SemaphoreType.DMA,
    ],
)(_sc_scatter_body)


def _wc(w1_ref, b1_ref, w2_ref, b2_ref):
    W = lax.dot_general(w1_ref[...], w2_ref[...], (((0,), (1,)), ((), ())),
                        preferred_element_type=jnp.float32)
    cv = lax.dot_general(b1_ref[...], w2_ref[...], (((1,), (1,)), ((), ())),
                         preferred_element_type=jnp.float32) + b2_ref[...]
    return W, cv


_W_SPECS = [pl.BlockSpec((D, D), lambda i: (0, 0)),
            pl.BlockSpec((1, D), lambda i: (0, 0)),
            pl.BlockSpec((D, D), lambda i: (0, 0)),
            pl.BlockSpec((1, D), lambda i: (0, 0))]


_BLK = 2000
_GRID = N // _BLK


def _mm1_body(x_ref, w1_ref, b1_ref, w2_ref, b2_ref, o_ref):
    W, cv = _wc(w1_ref, b1_ref, w2_ref, b2_ref)
    o_ref[...] = jnp.dot(x_ref[...], W,
                         preferred_element_type=jnp.float32) + cv


def _mm1(x, W1, b1, W2, b2):
    return pl.pallas_call(
        _mm1_body,
        grid=(_GRID,),
        in_specs=[pl.BlockSpec((_BLK, D), lambda i: (i, 0))] + _W_SPECS,
        out_specs=pl.BlockSpec((_BLK, D), lambda i: (i, 0)),
        out_shape=jax.ShapeDtypeStruct((N, D), jnp.float32),
    )(x, W1, b1[None, :], W2, b2[None, :])


def _mm2_body(p_ref, t_ref, w1_ref, b1_ref, w2_ref, b2_ref, o_ref):
    W, cv = _wc(w1_ref, b1_ref, w2_ref, b2_ref)
    h = p_ref[0] + p_ref[1] - t_ref[...]
    o_ref[...] = jnp.dot(h, W, preferred_element_type=jnp.float32) + cv


def _mm2(parts, t, W1, b1, W2, b2):
    return pl.pallas_call(
        _mm2_body,
        grid=(_GRID,),
        in_specs=[pl.BlockSpec((NC, _BLK, D), lambda i: (0, i, 0)),
                  pl.BlockSpec((_BLK, D), lambda i: (i, 0))] + _W_SPECS,
        out_specs=pl.BlockSpec((_BLK, D), lambda i: (i, 0)),
        out_shape=jax.ShapeDtypeStruct((N, D), jnp.float32),
    )(parts, t, W1, b1[None, :], W2, b2[None, :])


def _final_body(p_ref, t_ref, o_ref):
    z = p_ref[0] + p_ref[1] - t_ref[...]
    m = jnp.max(z, axis=1, keepdims=True)
    e = jnp.exp(z - m)
    o_ref[...] = (z - m) - jnp.log(jnp.sum(e, axis=1, keepdims=True))


def _final(parts, t):
    return pl.pallas_call(
        _final_body,
        grid=(_GRID,),
        in_specs=[pl.BlockSpec((NC, _BLK, D), lambda i: (0, i, 0)),
                  pl.BlockSpec((_BLK, D), lambda i: (i, 0))],
        out_specs=pl.BlockSpec((_BLK, D), lambda i: (i, 0)),
        out_shape=jax.ShapeDtypeStruct((N, D), jnp.float32),
    )(parts, t)


def kernel(x, edge_index, W1, b1, W2, b2):
    t1 = _mm1(x, W1, b1, W2, b2)
    edge_flat = edge_index.reshape(2 * E)
    parts1 = _sc_scatter(t1, edge_flat)
    t2 = _mm2(parts1, t1, W1, b1, W2, b2)
    parts2 = _sc_scatter(t2, edge_flat)
    return _final(parts2, t2)
